# baseline (speedup 1.0000x reference)
import functools

import jax
import jax.numpy as jnp
import numpy as np
from jax import lax
from jax.experimental import pallas as pl
from jax.experimental.pallas import tpu as pltpu
from jax.experimental.pallas import tpu_sc as plsc

_B = 2
_M = 60
_NNEG = 100
_TEMP = 0.1
_NOUT = (1 + _NNEG) * _B * _M
_NPAD = 12288

def _pack_patches(source):
    s320 = source.reshape(_B, 125, 320)
    nxt = jnp.concatenate(
        [s320[:, 1:, :10], jnp.zeros((_B, 1, 10), jnp.float32)], axis=1)
    g = jnp.concatenate([s320, nxt], axis=2)
    p4 = jnp.stack(
        [jnp.concatenate([g[:, :, 5 * (4 * gg + c):5 * (4 * gg + c) + 10]
                          for c in range(4)], axis=-1)
         for gg in range(16)], axis=1)
    p4 = jnp.pad(p4, ((0, 0), (0, 0), (0, 3), (0, 0)))
    return p4.reshape(_B, 2048, 40)


def _neg_index_pattern():
    tszs = jnp.repeat(jnp.arange(_M), _NNEG)
    neg = jax.random.randint(jax.random.key(42), (_B, _NNEG * _M), 0, _M - 1)
    neg = jnp.where(neg >= tszs[None, :], neg + 1, neg)
    neg = neg + jnp.arange(_B)[:, None] * 64
    negpart = jnp.transpose(neg.reshape(_B, _M, _NNEG), (2, 0, 1)).reshape(-1)
    ar = jnp.arange(_B * _M)
    pos = (ar // _M) * 64 + ar % _M
    flat = jnp.concatenate([pos, negpart])
    return jnp.pad(flat, (0, _NPAD - _NOUT)).astype(jnp.int32)


def _gelu(x):
    return jax.nn.gelu(x)


def _phase_conv(in_ref, out_ref, w_ref, p_out, ksize):
    f32 = jnp.float32
    w = w_ref[...]

    for p in range(p_out):
        a = jnp.dot(in_ref[pl.ds(256 * p, 128)], w[0],
                    preferred_element_type=f32)
        a = a + jnp.dot(in_ref[pl.ds(256 * p + 128, 128)], w[1],
                        preferred_element_type=f32)
        if ksize == 3:
            if p < p_out - 1:
                a = a + jnp.dot(in_ref[pl.ds(256 * p + 256, 128)], w[2],
                                preferred_element_type=f32)
            else:
                x0 = in_ref[pl.ds(0, 128)]
                x0s = jnp.concatenate(
                    [x0[1:], jnp.zeros((1, 512), jnp.bfloat16)], axis=0)
                a = a + jnp.dot(x0s, w[2], preferred_element_type=f32)
        out_ref[pl.ds(128 * p, 128)] = _gelu(a).astype(jnp.bfloat16)


def _mega_body(p_ref, w0_ref, gs_ref, gb_ref, w1_ref, w2_ref, w3_ref, w4_ref,
               w5_ref, w6_ref, lns_ref, lnb_ref, pq_ref, pqb_ref, me_ref,
               fw_ref, fb_ref, oh_ref, o_ref, sa_ref, sb_ref, sc_ref):
    f32 = jnp.float32
    w0 = w0_ref[...]
    for b in range(_B):
        s4 = jnp.zeros((1, 2048), f32)
        ss4 = jnp.zeros((1, 2048), f32)
        for g in range(16):
            x4 = jnp.dot(p_ref[b, pl.ds(g * 128, 128)], w0,
                         preferred_element_type=f32)
            s4 = s4 + jnp.sum(x4, axis=0, keepdims=True)
            ss4 = ss4 + jnp.sum(x4 * x4, axis=0, keepdims=True)
            for c in range(4):
                sa_ref[pl.ds(128 * (4 * g + c), 128)] = \
                    x4[:, 512 * c:512 * c + 512]
        s = jnp.sum(s4.reshape(4, 512), axis=0, keepdims=True)
        ss = jnp.sum(ss4.reshape(4, 512), axis=0, keepdims=True)
        last = sa_ref[pl.ds(8188, 1)]
        s = s - last
        ss = ss - last * last
        m = s / 7999.0
        v = ss / 7999.0 - m * m
        sc = lax.rsqrt(v + 1e-5) * gs_ref[...]
        for t in range(32):
            x = sa_ref[pl.ds(t * 256, 256)]
            sb_ref[pl.ds(t * 256, 256)] = \
                _gelu((x - m) * sc + gb_ref[...]).astype(jnp.bfloat16)
        _phase_conv(sb_ref, sc_ref, w1_ref, 32, 3)
        _phase_conv(sc_ref, sb_ref, w2_ref, 16, 3)
        _phase_conv(sb_ref, sc_ref, w3_ref, 8, 3)
        _phase_conv(sc_ref, sb_ref, w4_ref, 4, 3)
        _phase_conv(sb_ref, sc_ref, w5_ref, 2, 2)
        x = jnp.dot(sc_ref[pl.ds(0, 128)], w6_ref[0],
                    preferred_element_type=f32)
        x = x + jnp.dot(sc_ref[pl.ds(128, 128)], w6_ref[1],
                        preferred_element_type=f32)
        x = _gelu(x)
        mm = jnp.mean(x, axis=-1, keepdims=True)
        d = x - mm
        vv = jnp.mean(d * d, axis=-1, keepdims=True)
        xl = d * lax.rsqrt(vv + 1e-5) * lns_ref[...] + lnb_ref[...]
        y = jnp.dot(xl, pq_ref[...], preferred_element_type=f32) + pqb_ref[...]
        xv = jnp.dot(me_ref[...], fw_ref[...],
                     preferred_element_type=f32) + fb_ref[...]
        xn = xv / (jnp.sqrt(jnp.sum(xv * xv)) + 1e-8)
        yn = y / (jnp.sqrt(jnp.sum(y * y, axis=-1, keepdims=True)) + 1e-8)
        l = jnp.sum(yn * xn, axis=-1) / _TEMP
        sel = jnp.sum(oh_ref[b] * l.reshape(1, 128), axis=-1)
        o_ref[b, 0] = jnp.concatenate([sel, jnp.zeros((4,), f32)])


def _sc_gather(sel_table, flatidx):
    info = plsc.get_sparse_core_info()
    nw = info.num_cores * info.num_subcores
    ch = _NPAD // nw
    mesh = plsc.VectorSubcoreMesh(core_axis_name="c", subcore_axis_name="s")

    @functools.partial(
        pl.kernel, mesh=mesh,
        out_type=jax.ShapeDtypeStruct((_NPAD,), jnp.float32),
        compiler_params=pltpu.CompilerParams(needs_layout_passes=False),
        scratch_types=[
            pltpu.VMEM((128,), jnp.float32),
            pltpu.VMEM((ch,), jnp.int32),
            pltpu.VMEM((ch,), jnp.float32),
        ])
    def k(tab_hbm, idx_hbm, out_hbm, tab_v, idx_v, out_v):
        wid = lax.axis_index("s") * info.num_cores + lax.axis_index("c")
        base = wid * ch
        pltpu.sync_copy(tab_hbm, tab_v)
        pltpu.sync_copy(idx_hbm.at[pl.ds(base, ch)], idx_v)
        for j in range(ch // 16):
            jdx = idx_v[pl.ds(j * 16, 16)]
            out_v[pl.ds(j * 16, 16)] = plsc.load_gather(tab_v, [jdx])
        pltpu.sync_copy(out_v, out_hbm.at[pl.ds(base, ch)])

    return k(sel_table, flatidx)


def kernel(source, masked_pos, conv_w0, conv_w1, conv_w2, conv_w3, conv_w4,
           conv_w5, conv_w6, gn_scale, gn_bias, ln_scale, ln_bias, proj_w,
           proj_b, mask_emb, projq_w, projq_b, final_w, final_b):
    f32 = jnp.float32
    p_pm = _pack_patches(source)
    w0r = conv_w0.reshape(10, 512)
    wbd = jnp.zeros((40, 2048), f32)
    for c in range(4):
        wbd = wbd.at[10 * c:10 * c + 10, 512 * c:512 * c + 512].set(w0r)

    l0 = pl.pallas_call(
        _mega_body,
        grid=(1,),
        in_specs=[
            pl.BlockSpec((_B, 2048, 40), lambda g: (0, 0, 0)),
            pl.BlockSpec((40, 2048), lambda g: (0, 0)),
            pl.BlockSpec((1, 512), lambda g: (0, 0)),
            pl.BlockSpec((1, 512), lambda g: (0, 0)),
            pl.BlockSpec((3, 512, 512), lambda g: (0, 0, 0)),
            pl.BlockSpec((3, 512, 512), lambda g: (0, 0, 0)),
            pl.BlockSpec((3, 512, 512), lambda g: (0, 0, 0)),
            pl.BlockSpec((3, 512, 512), lambda g: (0, 0, 0)),
            pl.BlockSpec((2, 512, 512), lambda g: (0, 0, 0)),
            pl.BlockSpec((2, 512, 512), lambda g: (0, 0, 0)),
            pl.BlockSpec((1, 512), lambda g: (0, 0)),
            pl.BlockSpec((1, 512), lambda g: (0, 0)),
            pl.BlockSpec((512, 256), lambda g: (0, 0)),
            pl.BlockSpec((1, 256), lambda g: (0, 0)),
            pl.BlockSpec((1, 768), lambda g: (0, 0)),
            pl.BlockSpec((768, 256), lambda g: (0, 0)),
            pl.BlockSpec((1, 256), lambda g: (0, 0)),
            pl.BlockSpec((_B, 60, 128), lambda g: (0, 0, 0)),
        ],
        out_specs=pl.BlockSpec((_B, 1, 64), lambda g: (0, 0, 0)),
        out_shape=jax.ShapeDtypeStruct((_B, 1, 64), f32),
        scratch_shapes=[
            pltpu.VMEM((8192, 512), f32),
            pltpu.VMEM((8192, 512), jnp.bfloat16),
            pltpu.VMEM((4096, 512), jnp.bfloat16),
        ],
    )(p_pm, wbd, gn_scale.reshape(1, 512), gn_bias.reshape(1, 512),
      conv_w1.astype(jnp.bfloat16), conv_w2.astype(jnp.bfloat16),
      conv_w3.astype(jnp.bfloat16), conv_w4.astype(jnp.bfloat16),
      conv_w5.astype(jnp.bfloat16), conv_w6.astype(jnp.bfloat16),
      ln_scale.reshape(1, 512), ln_bias.reshape(1, 512),
      projq_w, projq_b.reshape(1, 256),
      mask_emb.reshape(1, 768), final_w, final_b.reshape(1, 256),
      jax.nn.one_hot(masked_pos, 128, dtype=f32))

    flat = _sc_gather(l0.reshape(_B * 64), _neg_index_pattern())
    return flat[:_NOUT].reshape(1 + _NNEG, _B, _M)


# speedup vs baseline: 1.0642x; 1.0642x over previous
import functools

import jax
import jax.numpy as jnp
import numpy as np
from jax import lax
from jax.experimental import pallas as pl
from jax.experimental.pallas import tpu as pltpu
from jax.experimental.pallas import tpu_sc as plsc

_B = 2
_M = 60
_NNEG = 100
_TEMP = 0.1
_NOUT = (1 + _NNEG) * _B * _M
_NPAD = 12288

def _pack_patches(source):
    s320 = source.reshape(_B, 125, 320)
    nxt = jnp.concatenate(
        [s320[:, 1:, :10], jnp.zeros((_B, 1, 10), jnp.float32)], axis=1)
    g = jnp.concatenate([s320, nxt], axis=2)
    p4 = jnp.stack(
        [jnp.concatenate([g[:, :, 5 * (4 * gg + c):5 * (4 * gg + c) + 10]
                          for c in range(4)], axis=-1)
         for gg in range(16)], axis=1)
    p4 = jnp.pad(p4, ((0, 0), (0, 0), (0, 3), (0, 0)))
    return p4.reshape(_B, 2048, 40)


def _neg_index_pattern():
    tszs = jnp.repeat(jnp.arange(_M), _NNEG)
    neg = jax.random.randint(jax.random.key(42), (_B, _NNEG * _M), 0, _M - 1)
    neg = jnp.where(neg >= tszs[None, :], neg + 1, neg)
    neg = neg + jnp.arange(_B)[:, None] * 64
    negpart = jnp.transpose(neg.reshape(_B, _M, _NNEG), (2, 0, 1)).reshape(-1)
    ar = jnp.arange(_B * _M)
    pos = (ar // _M) * 64 + ar % _M
    flat = jnp.concatenate([pos, negpart])
    return jnp.pad(flat, (0, _NPAD - _NOUT)).astype(jnp.int32)


def _gelu(x):
    return jax.nn.gelu(x)


def _phase_conv(in_ref, out_ref, w_ref, p_out, ksize):
    f32 = jnp.float32
    w = w_ref[...]

    for p in range(p_out):
        a = jnp.dot(in_ref[pl.ds(256 * p, 128)], w[0],
                    preferred_element_type=f32)
        a = a + jnp.dot(in_ref[pl.ds(256 * p + 128, 128)], w[1],
                        preferred_element_type=f32)
        if ksize == 3:
            if p < p_out - 1:
                a = a + jnp.dot(in_ref[pl.ds(256 * p + 256, 128)], w[2],
                                preferred_element_type=f32)
            else:
                x0 = in_ref[pl.ds(0, 128)]
                x0s = jnp.concatenate(
                    [x0[1:], jnp.zeros((1, 512), f32)], axis=0)
                a = a + jnp.dot(x0s, w[2], preferred_element_type=f32)
        out_ref[pl.ds(128 * p, 128)] = _gelu(a)


def _mega_body(p_ref, w0_ref, gs_ref, gb_ref, w1_ref, w2_ref, w3_ref, w4_ref,
               w5_ref, w6_ref, lns_ref, lnb_ref, pq_ref, pqb_ref, me_ref,
               fw_ref, fb_ref, oh_ref, o_ref, sa_ref, sb_ref):
    f32 = jnp.float32
    w0 = w0_ref[...]
    for b in range(_B):
        s4 = jnp.zeros((1, 2048), f32)
        ss4 = jnp.zeros((1, 2048), f32)
        for g in range(16):
            x4 = jnp.dot(p_ref[b, pl.ds(g * 128, 128)], w0,
                         preferred_element_type=f32)
            s4 = s4 + jnp.sum(x4, axis=0, keepdims=True)
            ss4 = ss4 + jnp.sum(x4 * x4, axis=0, keepdims=True)
            for c in range(4):
                sa_ref[pl.ds(128 * (4 * g + c), 128)] = \
                    x4[:, 512 * c:512 * c + 512]
        s = jnp.sum(s4.reshape(4, 512), axis=0, keepdims=True)
        ss = jnp.sum(ss4.reshape(4, 512), axis=0, keepdims=True)
        last = sa_ref[pl.ds(8188, 1)]
        s = s - last
        ss = ss - last * last
        m = s / 7999.0
        v = ss / 7999.0 - m * m
        sc = lax.rsqrt(v + 1e-5) * gs_ref[...]
        for t in range(32):
            x = sa_ref[pl.ds(t * 256, 256)]
            sb_ref[pl.ds(t * 256, 256)] = _gelu((x - m) * sc + gb_ref[...])
        _phase_conv(sb_ref, sa_ref, w1_ref, 32, 3)
        _phase_conv(sa_ref, sb_ref, w2_ref, 16, 3)
        _phase_conv(sb_ref, sa_ref, w3_ref, 8, 3)
        _phase_conv(sa_ref, sb_ref, w4_ref, 4, 3)
        _phase_conv(sb_ref, sa_ref, w5_ref, 2, 2)
        x = jnp.dot(sa_ref[pl.ds(0, 128)], w6_ref[0],
                    preferred_element_type=f32)
        x = x + jnp.dot(sa_ref[pl.ds(128, 128)], w6_ref[1],
                        preferred_element_type=f32)
        x = _gelu(x)
        mm = jnp.mean(x, axis=-1, keepdims=True)
        d = x - mm
        vv = jnp.mean(d * d, axis=-1, keepdims=True)
        xl = d * lax.rsqrt(vv + 1e-5) * lns_ref[...] + lnb_ref[...]
        y = jnp.dot(xl, pq_ref[...], preferred_element_type=f32) + pqb_ref[...]
        xv = jnp.dot(me_ref[...], fw_ref[...],
                     preferred_element_type=f32) + fb_ref[...]
        xn = xv / (jnp.sqrt(jnp.sum(xv * xv)) + 1e-8)
        yn = y / (jnp.sqrt(jnp.sum(y * y, axis=-1, keepdims=True)) + 1e-8)
        l = jnp.sum(yn * xn, axis=-1) / _TEMP
        sel = jnp.sum(oh_ref[b] * l.reshape(1, 128), axis=-1)
        o_ref[b, 0] = jnp.concatenate([sel, jnp.zeros((4,), f32)])


def _sc_gather(sel_table, flatidx):
    info = plsc.get_sparse_core_info()
    nw = info.num_cores * info.num_subcores
    ch = _NPAD // nw
    mesh = plsc.VectorSubcoreMesh(core_axis_name="c", subcore_axis_name="s")

    @functools.partial(
        pl.kernel, mesh=mesh,
        out_type=jax.ShapeDtypeStruct((_NPAD,), jnp.float32),
        compiler_params=pltpu.CompilerParams(needs_layout_passes=False),
        scratch_types=[
            pltpu.VMEM((128,), jnp.float32),
            pltpu.VMEM((ch,), jnp.int32),
            pltpu.VMEM((ch,), jnp.float32),
        ])
    def k(tab_hbm, idx_hbm, out_hbm, tab_v, idx_v, out_v):
        wid = lax.axis_index("s") * info.num_cores + lax.axis_index("c")
        base = wid * ch
        pltpu.sync_copy(tab_hbm, tab_v)
        pltpu.sync_copy(idx_hbm.at[pl.ds(base, ch)], idx_v)
        for j in range(ch // 16):
            jdx = idx_v[pl.ds(j * 16, 16)]
            out_v[pl.ds(j * 16, 16)] = plsc.load_gather(tab_v, [jdx])
        pltpu.sync_copy(out_v, out_hbm.at[pl.ds(base, ch)])

    return k(sel_table, flatidx)


def kernel(source, masked_pos, conv_w0, conv_w1, conv_w2, conv_w3, conv_w4,
           conv_w5, conv_w6, gn_scale, gn_bias, ln_scale, ln_bias, proj_w,
           proj_b, mask_emb, projq_w, projq_b, final_w, final_b):
    f32 = jnp.float32
    p_pm = _pack_patches(source)
    w0r = conv_w0.reshape(10, 512)
    wbd = jnp.zeros((40, 2048), f32)
    for c in range(4):
        wbd = wbd.at[10 * c:10 * c + 10, 512 * c:512 * c + 512].set(w0r)

    l0 = pl.pallas_call(
        _mega_body,
        grid=(1,),
        in_specs=[
            pl.BlockSpec((_B, 2048, 40), lambda g: (0, 0, 0)),
            pl.BlockSpec((40, 2048), lambda g: (0, 0)),
            pl.BlockSpec((1, 512), lambda g: (0, 0)),
            pl.BlockSpec((1, 512), lambda g: (0, 0)),
            pl.BlockSpec((3, 512, 512), lambda g: (0, 0, 0)),
            pl.BlockSpec((3, 512, 512), lambda g: (0, 0, 0)),
            pl.BlockSpec((3, 512, 512), lambda g: (0, 0, 0)),
            pl.BlockSpec((3, 512, 512), lambda g: (0, 0, 0)),
            pl.BlockSpec((2, 512, 512), lambda g: (0, 0, 0)),
            pl.BlockSpec((2, 512, 512), lambda g: (0, 0, 0)),
            pl.BlockSpec((1, 512), lambda g: (0, 0)),
            pl.BlockSpec((1, 512), lambda g: (0, 0)),
            pl.BlockSpec((512, 256), lambda g: (0, 0)),
            pl.BlockSpec((1, 256), lambda g: (0, 0)),
            pl.BlockSpec((1, 768), lambda g: (0, 0)),
            pl.BlockSpec((768, 256), lambda g: (0, 0)),
            pl.BlockSpec((1, 256), lambda g: (0, 0)),
            pl.BlockSpec((_B, 60, 128), lambda g: (0, 0, 0)),
        ],
        out_specs=pl.BlockSpec((_B, 1, 64), lambda g: (0, 0, 0)),
        out_shape=jax.ShapeDtypeStruct((_B, 1, 64), f32),
        scratch_shapes=[
            pltpu.VMEM((8192, 512), f32),
            pltpu.VMEM((8192, 512), f32),
        ],
    )(p_pm, wbd, gn_scale.reshape(1, 512), gn_bias.reshape(1, 512),
      conv_w1, conv_w2, conv_w3, conv_w4, conv_w5, conv_w6,
      ln_scale.reshape(1, 512), ln_bias.reshape(1, 512),
      projq_w, projq_b.reshape(1, 256),
      mask_emb.reshape(1, 768), final_w, final_b.reshape(1, 256),
      jax.nn.one_hot(masked_pos, 128, dtype=f32))

    flat = _sc_gather(l0.reshape(_B * 64), _neg_index_pattern())
    return flat[:_NOUT].reshape(1 + _NNEG, _B, _M)
